# Initial kernel scaffold; baseline (speedup 1.0000x reference)
import jax
import jax.numpy as jnp
from jax.experimental import pallas as pl


def kernel(x, edge_index, edge_attr, batch, edge_emb0, W0, Wedge0, att_src0, att_dst0, att_edge0, bias0, gamma0, beta0, edge_emb1, W1, Wedge1, att_src1, att_dst1, att_edge1, bias1, gamma1, beta1, fc_W1, fc_b1, fc_W2, fc_b2):
    raise NotImplementedError("write your pallas kernel here")


# speedup vs baseline: 23.4092x; 23.4092x over previous
import functools

import jax
import jax.numpy as jnp
from jax import lax
from jax.experimental import pallas as pl
from jax.experimental.pallas import tpu as pltpu
from jax.experimental.pallas import tpu_sc as plsc

N = 10000
E = 320000
D = 128
G = 64

NPAD = 10080
DPAD = 144
NC = 2
NS = 16
L = 16
C = 128
QC = 81
TPE = QC * C
EPAD = NC * NS * TPE
RPT = NPAD // (NC * NS) * NC


def _prep_body(x_ref, w_ref, asrc_ref, adst_ref, hpad_ref, al_ref, ar_ref):
    x = x_ref[...]
    h = jnp.dot(x, w_ref[...], preferred_element_type=jnp.float32, precision=lax.Precision.HIGHEST)
    al = jnp.dot(h, asrc_ref[...], preferred_element_type=jnp.float32, precision=lax.Precision.HIGHEST)
    ar = jnp.dot(h, adst_ref[...], preferred_element_type=jnp.float32, precision=lax.Precision.HIGHEST)
    hp = jnp.concatenate(
        [h, jnp.ones((N, 1), jnp.float32), jnp.zeros((N, DPAD - D - 1), jnp.float32)], axis=1)
    hpad_ref[...] = jnp.concatenate([hp, jnp.zeros((NPAD - N, DPAD), jnp.float32)], axis=0)
    zpad = jnp.zeros((NPAD - N, 1), jnp.float32)
    al_ref[...] = jnp.concatenate([al, zpad], axis=0)
    ar_ref[...] = jnp.concatenate([ar, zpad], axis=0)


_prep_call = pl.pallas_call(
    _prep_body,
    out_shape=(
        jax.ShapeDtypeStruct((NPAD, DPAD), jnp.float32),
        jax.ShapeDtypeStruct((NPAD, 1), jnp.float32),
        jax.ShapeDtypeStruct((NPAD, 1), jnp.float32),
    ),
)


def _w8_body(attr_ref, e0_ref, we0_ref, ae0_ref, e1_ref, we1_ref, ae1_ref, out_ref):
    a = attr_ref[...]
    cnt = [jnp.sum((a == k).astype(jnp.float32)) for k in range(4)]
    rows = []
    for e_ref, we_ref, ae_ref in ((e0_ref, we0_ref, ae0_ref), (e1_ref, we1_ref, ae1_ref)):
        t = jnp.dot(e_ref[...], we_ref[...], preferred_element_type=jnp.float32, precision=lax.Precision.HIGHEST)
        w4 = jnp.dot(t, ae_ref[...], preferred_element_type=jnp.float32, precision=lax.Precision.HIGHEST)
        mean_ae = (cnt[0] * w4[0, 0] + cnt[1] * w4[1, 0]
                   + cnt[2] * w4[2, 0] + cnt[3] * w4[3, 0]) / jnp.float32(E)
        row = jnp.concatenate(
            [w4[:, 0], mean_ae[None], jnp.full((3,), -jnp.inf, jnp.float32)])
        rows.append(row[None, :])
    out_ref[...] = jnp.concatenate(rows, axis=0)


_w8_call = pl.pallas_call(
    _w8_body,
    out_shape=jax.ShapeDtypeStruct((2, 8), jnp.float32),
)


def _post_body(acc_ref, bias_ref, gamma_ref, beta_ref, y_ref):
    a = acc_ref[:NPAD, :] + acc_ref[NPAD:, :]
    num = a[:N, :D]
    den = a[:N, D:D + 1]
    o = num / (den + jnp.float32(1e-16)) + bias_ref[...]
    m = jnp.mean(o, axis=0, keepdims=True)
    d = o - m
    v = jnp.mean(d * d, axis=0, keepdims=True)
    y = d / jnp.sqrt(v + jnp.float32(1e-5)) * gamma_ref[...] + beta_ref[...]
    y_ref[...] = jnp.maximum(y, 0.0)


_post_call = pl.pallas_call(
    _post_body,
    out_shape=jax.ShapeDtypeStruct((N, D), jnp.float32),
)


def _final_body(x_ref, b_ref, w1_ref, b1_ref, w2_ref, b2_ref, out_ref):
    x = x_ref[...]
    b = b_ref[:N, :]
    p = (b == lax.broadcasted_iota(jnp.int32, (N, G), 1)).astype(jnp.float32)
    s = lax.dot_general(p, x, (((0,), (0,)), ((), ())),
                        preferred_element_type=jnp.float32, precision=lax.Precision.HIGHEST)
    cnt = lax.dot_general(p, jnp.ones((N, 1), jnp.float32), (((0,), (0,)), ((), ())),
                          preferred_element_type=jnp.float32, precision=lax.Precision.HIGHEST)
    hg = s / jnp.maximum(cnt, 1.0)
    z = jnp.maximum(jnp.dot(hg, w1_ref[...], preferred_element_type=jnp.float32, precision=lax.Precision.HIGHEST)
                    + b1_ref[...], 0.0)
    out_ref[...] = jnp.dot(z, w2_ref[...], preferred_element_type=jnp.float32, precision=lax.Precision.HIGHEST) + b2_ref[...]


_final_call = pl.pallas_call(
    _final_body,
    out_shape=jax.ShapeDtypeStruct((G, 1), jnp.float32),
)


@functools.partial(
    pl.kernel,
    out_type=jax.ShapeDtypeStruct((NC * NPAD, DPAD), jnp.float32),
    mesh=plsc.VectorSubcoreMesh(core_axis_name="c", subcore_axis_name="s"),
    compiler_params=pltpu.CompilerParams(
        needs_layout_passes=False, use_tc_tiling_on_sc=False),
    scratch_types=[
        pltpu.VMEM((NPAD,), jnp.float32),
        pltpu.VMEM((NPAD,), jnp.float32),
        pltpu.VMEM((8,), jnp.float32),
        pltpu.VMEM((C,), jnp.int32),
        pltpu.VMEM((C,), jnp.int32),
        pltpu.VMEM((C,), jnp.int32),
        pltpu.VMEM((C, DPAD), jnp.float32),
        pltpu.VMEM((C,), jnp.float32),
        pltpu.VMEM_SHARED((NPAD, DPAD), jnp.float32),
        pltpu.SemaphoreType.DMA,
    ],
)
def _sc_edge(hpad_h, al_h, ar_h, w8_h, src_h, dst_h, attr_h, zz_h, out_h,
             al_v, ar_v, w8_v, sidx, didx, aidx, rows, exv, acc, sem):
    cid = lax.axis_index("c")
    sub = lax.axis_index("s")
    wid = cid * NS + sub

    pltpu.sync_copy(al_h, al_v)
    pltpu.sync_copy(ar_h, ar_v)
    pltpu.sync_copy(w8_h, w8_v)
    pltpu.sync_copy(zz_h, acc.at[pl.ds(sub * RPT, RPT)])
    plsc.subcore_barrier()

    def chunk(ch, carry):
        base = wid * TPE + ch * C
        cps = pltpu.async_copy(src_h.at[pl.ds(base, C)], sidx, sem)
        cpd = pltpu.async_copy(dst_h.at[pl.ds(base, C)], didx, sem)
        cpa = pltpu.async_copy(attr_h.at[pl.ds(base, C)], aidx, sem)
        cps.wait()
        cpd.wait()
        cpa.wait()
        pltpu.async_copy(hpad_h.at[sidx], rows, sem).wait()
        for g in range(C // L):
            sl = pl.ds(g * L, L)
            s16 = sidx[sl]
            d16 = didx[sl]
            a16 = aidx[sl]
            av = plsc.load_gather(al_v, [s16])
            rv = plsc.load_gather(ar_v, [d16])
            wv = plsc.load_gather(w8_v, [a16])
            alp = av + rv + wv
            alp = jnp.where(alp >= 0, alp, alp * jnp.float32(0.2))
            exv[sl] = jnp.exp(alp)

        def scale(g2, carry2):
            ex16 = exv[pl.ds(g2 * L, L)]
            for i in range(L):
                e = ex16[i]
                r = g2 * L + i
                for k in range(DPAD // L):
                    s2 = pl.ds(k * L, L)
                    rows[r, s2] = rows[r, s2] * e
            return carry2

        lax.fori_loop(0, C // L, scale, 0)
        pltpu.sync_copy(rows, acc.at[didx], add=True)
        return carry

    lax.fori_loop(0, QC, chunk, 0)
    plsc.subcore_barrier()
    pltpu.sync_copy(acc.at[pl.ds(sub * RPT, RPT)],
                    out_h.at[pl.ds(cid * NPAD + sub * RPT, RPT)])


def kernel(x, edge_index, edge_attr, batch,
           edge_emb0, W0, Wedge0, att_src0, att_dst0, att_edge0, bias0, gamma0, beta0,
           edge_emb1, W1, Wedge1, att_src1, att_dst1, att_edge1, bias1, gamma1, beta1,
           fc_W1, fc_b1, fc_W2, fc_b2):
    i32 = jnp.int32
    loop = jnp.arange(N, dtype=i32)
    npad_e = EPAD - E - N
    srcf = jnp.concatenate([edge_index[0], loop, jnp.full((npad_e,), NPAD - 1, i32)])
    dstf = jnp.concatenate([edge_index[1], loop, jnp.full((npad_e,), NPAD - 1, i32)])
    attrf = jnp.concatenate([edge_attr, jnp.full((N,), 4, i32), jnp.full((npad_e,), 5, i32)])
    zeros_rpt = jnp.zeros((RPT, DPAD), jnp.float32)

    w8s = _w8_call(edge_attr.reshape(E // D, D),
                   edge_emb0, Wedge0, att_edge0.reshape(D, 1),
                   edge_emb1, Wedge1, att_edge1.reshape(D, 1))

    params = (
        (W0, att_src0, att_dst0, bias0, gamma0, beta0),
        (W1, att_src1, att_dst1, bias1, gamma1, beta1),
    )
    h = x
    for l, (W, a_s, a_d, bias, gamma, beta) in enumerate(params):
        hpad, al2, ar2 = _prep_call(h, W, a_s.reshape(D, 1), a_d.reshape(D, 1))
        acc = _sc_edge(hpad, al2.reshape(NPAD), ar2.reshape(NPAD), w8s[l],
                       srcf, dstf, attrf, zeros_rpt)
        h = _post_call(acc, bias.reshape(1, D), gamma.reshape(1, D), beta.reshape(1, D))

    bpad = jnp.concatenate([batch, jnp.full((NPAD - N,), G, i32)]).reshape(NPAD, 1)
    return _final_call(h, bpad, fc_W1, fc_b1.reshape(1, 2 * D),
                       fc_W2, fc_b2.reshape(1, 1))
